# baseline (speedup 1.0000x reference)
import functools

import jax
import jax.numpy as jnp
from jax import lax
from jax.experimental import pallas as pl
from jax.experimental.pallas import tpu as pltpu
from jax.experimental.pallas import tpu_sc as plsc

B = 16384
D = 32
L = 16
N_USERS = 1000000
N_TAIL = N_USERS % 128
TAIL0 = N_USERS - N_TAIL
U_PAD = TAIL0 + 128

_info = plsc.get_sparse_core_info()
_NC, _NS = _info.num_cores, _info.num_subcores
NW = _NC * _NS
B_PER_W = B // NW
N_GROUPS = B_PER_W // L

WIN = 256
LANES_PER_W = 31232
N_WIN = 124


def _conv_body(uf3_hbm, uf_tail_hbm, urows_hbm,
               sbuf0, sbuf1, wbuf0, wbuf1,
               sin0, sin1, sout0, sout1):
    wid = lax.axis_index("s") * _NC + lax.axis_index("c")
    lane_base = wid * LANES_PER_W

    sbufs = (sbuf0, sbuf1)
    wbufs = (wbuf0, wbuf1)
    sins = (sin0, sin1)
    souts = (sout0, sout1)

    k_idx_lo = lax.iota(jnp.int32, L) // 8
    r_idx_lo = lax.iota(jnp.int32, L) % 8
    k_idx_hi = k_idx_lo + 2
    r_idx_hi = r_idx_lo

    def start_in(win, slot):
        lane0 = pl.multiple_of(lane_base + win * WIN, 128)
        return pltpu.async_copy(
            uf3_hbm.at[slice(None), slice(None), pl.ds(lane0, WIN)],
            sbufs[slot], sins[slot])

    def transpose_window(slot):
        sbuf = sbufs[slot]
        wbuf = wbufs[slot]

        def item(i, carry):
            lane_i = jnp.full((L,), i, jnp.int32)
            lo = plsc.load_gather(sbuf, [k_idx_lo, r_idx_lo, lane_i])
            hi = plsc.load_gather(sbuf, [k_idx_hi, r_idx_hi, lane_i])
            wbuf[i, pl.ds(0, L)] = lo
            wbuf[i, pl.ds(L, L)] = hi
            return carry

        lax.fori_loop(0, WIN, item, 0)

    def start_out(win, slot):
        lane0 = pl.multiple_of(lane_base + win * WIN, 128)
        return pltpu.async_copy(
            wbufs[slot], urows_hbm.at[pl.ds(lane0, WIN)], souts[slot])

    cp_in = [start_in(0, 0), start_in(1, 1)]
    cp_out = [None, None]
    for win in range(N_WIN):
        slot = win % 2
        cp_in[slot].wait()
        if cp_out[slot] is not None:
            cp_out[slot].wait()
        transpose_window(slot)
        cp_out[slot] = start_out(win, slot)
        if win + 2 < N_WIN:
            cp_in[slot] = start_in(win + 2, slot)
    cp_out[0].wait()
    cp_out[1].wait()

    @pl.when(wid == 0)
    def _():
        pltpu.sync_copy(uf_tail_hbm, wbuf0.at[pl.ds(0, 128)])
        pltpu.sync_copy(wbuf0.at[pl.ds(0, 128)], urows_hbm.at[pl.ds(TAIL0, 128)])


@functools.partial(jax.jit, donate_argnums=())
def _convert(uf3, uf_tail):
    mesh = plsc.VectorSubcoreMesh(core_axis_name="c", subcore_axis_name="s")
    k = functools.partial(
        pl.kernel,
        mesh=mesh,
        out_type=jax.ShapeDtypeStruct((U_PAD, D), jnp.float32),
        compiler_params=pltpu.CompilerParams(
            needs_layout_passes=False,
        ),
        scratch_types=[
            pltpu.VMEM((4, 8, WIN), jnp.float32),
            pltpu.VMEM((4, 8, WIN), jnp.float32),
            pltpu.VMEM((WIN, D), jnp.float32),
            pltpu.VMEM((WIN, D), jnp.float32),
            pltpu.SemaphoreType.DMA,
            pltpu.SemaphoreType.DMA,
            pltpu.SemaphoreType.DMA,
            pltpu.SemaphoreType.DMA,
        ],
    )(_conv_body)
    return k(uf3, uf_tail)


def _dot_body(uid_hbm, aid_hbm, uf_hbm, gf_hbm, ub_hbm, gb_hbm, out_hbm,
              uid_v, aid_v, urow_v, grow_v, ubias_v, gbias_v, out_v,
              sem_u, sem_g, sem_ub, sem_gb):
    wid = lax.axis_index("s") * _NC + lax.axis_index("c")
    base = wid * B_PER_W

    pltpu.sync_copy(uid_hbm.at[pl.ds(base, B_PER_W)], uid_v)
    pltpu.sync_copy(aid_hbm.at[pl.ds(base, B_PER_W)], aid_v)

    copies = [
        pltpu.async_copy(uf_hbm.at[uid_v], urow_v, sem_u),
        pltpu.async_copy(gf_hbm.at[aid_v], grow_v, sem_g),
        pltpu.async_copy(ub_hbm.at[uid_v], ubias_v, sem_ub),
        pltpu.async_copy(gb_hbm.at[aid_v], gbias_v, sem_gb),
    ]
    for cp in copies:
        cp.wait()

    lanes = lax.iota(jnp.int32, L)

    def group(grp, carry):
        row0 = grp * L
        row_idx = lanes + row0
        acc = ubias_v[pl.ds(row0, L)] + gbias_v[pl.ds(row0, L)]
        for j in range(D):
            col_idx = jnp.full((L,), j, jnp.int32)
            u = plsc.load_gather(urow_v, [row_idx, col_idx])
            g = plsc.load_gather(grow_v, [row_idx, col_idx])
            acc = acc + u * g
        out_v[pl.ds(row0, L)] = jnp.maximum(acc, 0.0)
        return carry

    lax.fori_loop(0, N_GROUPS, group, 0)
    pltpu.sync_copy(out_v, out_hbm.at[pl.ds(base, B_PER_W)])


@jax.jit
def _run(user_ids, app_ids, urows, game_factors, user_bias, game_bias):
    mesh = plsc.VectorSubcoreMesh(core_axis_name="c", subcore_axis_name="s")
    k = functools.partial(
        pl.kernel,
        mesh=mesh,
        out_type=jax.ShapeDtypeStruct((B,), jnp.float32),
        compiler_params=pltpu.CompilerParams(
            use_tc_tiling_on_sc=False,
            needs_layout_passes=False,
        ),
        scratch_types=[
            pltpu.VMEM((B_PER_W,), jnp.int32),
            pltpu.VMEM((B_PER_W,), jnp.int32),
            pltpu.VMEM((B_PER_W, D), jnp.float32),
            pltpu.VMEM((B_PER_W, D), jnp.float32),
            pltpu.VMEM((B_PER_W,), jnp.float32),
            pltpu.VMEM((B_PER_W,), jnp.float32),
            pltpu.VMEM((B_PER_W,), jnp.float32),
            pltpu.SemaphoreType.DMA,
            pltpu.SemaphoreType.DMA,
            pltpu.SemaphoreType.DMA,
            pltpu.SemaphoreType.DMA,
        ],
    )(_dot_body)
    return k(user_ids, app_ids, urows, game_factors, user_bias, game_bias)


def kernel(user_ids, app_ids, user_factors, game_factors, user_bias, game_bias):
    uf3 = user_factors.T.reshape(4, 8, N_USERS)
    uf_tail = jnp.concatenate(
        [user_factors[TAIL0:], jnp.zeros((128 - N_TAIL, D), jnp.float32)])
    urows = _convert(uf3, uf_tail)
    return _run(user_ids, app_ids, urows, game_factors, user_bias, game_bias)


# speedup vs baseline: 1.0630x; 1.0630x over previous
import functools

import jax
import jax.numpy as jnp
from jax import lax
from jax.experimental import pallas as pl
from jax.experimental.pallas import tpu as pltpu
from jax.experimental.pallas import tpu_sc as plsc

B = 16384
D = 32
L = 16
N_USERS = 1000000
N_TAIL = N_USERS % 128
TAIL0 = N_USERS - N_TAIL
U_PAD = TAIL0 + 128

_info = plsc.get_sparse_core_info()
_NC, _NS = _info.num_cores, _info.num_subcores
NW = _NC * _NS
B_PER_W = B // NW
N_GROUPS = B_PER_W // L

WIN = 256
LANES_PER_W = 31232
N_WIN = 124


def _conv_body(uf3_hbm, uf_tail_hbm, urows_hbm,
               sbuf0, sbuf1, wbuf0, wbuf1,
               sin0, sin1, sout0, sout1):
    wid = lax.axis_index("s") * _NC + lax.axis_index("c")
    lane_base = wid * LANES_PER_W

    sbufs = (sbuf0, sbuf1)
    wbufs = (wbuf0, wbuf1)
    sins = (sin0, sin1)
    souts = (sout0, sout1)

    lanes = lax.iota(jnp.int32, L)

    def start_in(win, slot):
        lane0 = pl.multiple_of(lane_base + win * WIN, 128)
        return pltpu.async_copy(
            uf3_hbm.at[slice(None), slice(None), pl.ds(lane0, WIN)],
            sbufs[slot], sins[slot])

    def transpose_window(slot):
        sbuf = sbufs[slot]
        wbuf = wbufs[slot]

        def group(g, carry):
            rows = g * L + lanes
            for c in range(D):
                v = plsc.load_gather(
                    sbuf,
                    [jnp.full((L,), c // 8, jnp.int32),
                     jnp.full((L,), c % 8, jnp.int32),
                     rows])
                plsc.store_scatter(
                    wbuf, [rows, jnp.full((L,), c, jnp.int32)], v)
            return carry

        lax.fori_loop(0, WIN // L, group, 0)

    def start_out(win, slot):
        lane0 = pl.multiple_of(lane_base + win * WIN, 128)
        return pltpu.async_copy(
            wbufs[slot], urows_hbm.at[pl.ds(lane0, WIN)], souts[slot])

    def drain_in(slot):
        pltpu.make_async_copy(
            uf3_hbm.at[slice(None), slice(None), pl.ds(0, WIN)],
            sbufs[slot], sins[slot]).wait()

    def drain_out(slot):
        pltpu.make_async_copy(
            wbufs[slot], urows_hbm.at[pl.ds(0, WIN)], souts[slot]).wait()

    start_in(0, 0)
    start_in(1, 1)

    def pair(p, carry):
        for b in range(2):
            win = 2 * p + b
            drain_in(b)

            @pl.when(p > 0)
            def _():
                drain_out(b)

            transpose_window(b)
            start_out(win, b)

            @pl.when(win + 2 < N_WIN)
            def _():
                start_in(win + 2, b)
        return carry

    lax.fori_loop(0, N_WIN // 2, pair, 0)
    drain_out(0)
    drain_out(1)

    @pl.when(wid == 0)
    def _():
        pltpu.sync_copy(uf_tail_hbm, wbuf0.at[pl.ds(0, 128)])
        pltpu.sync_copy(wbuf0.at[pl.ds(0, 128)], urows_hbm.at[pl.ds(TAIL0, 128)])


@functools.partial(jax.jit, donate_argnums=())
def _convert(uf3, uf_tail):
    mesh = plsc.VectorSubcoreMesh(core_axis_name="c", subcore_axis_name="s")
    k = functools.partial(
        pl.kernel,
        mesh=mesh,
        out_type=jax.ShapeDtypeStruct((U_PAD, D), jnp.float32),
        compiler_params=pltpu.CompilerParams(
            needs_layout_passes=False,
        ),
        scratch_types=[
            pltpu.VMEM((4, 8, WIN), jnp.float32),
            pltpu.VMEM((4, 8, WIN), jnp.float32),
            pltpu.VMEM((WIN, D), jnp.float32),
            pltpu.VMEM((WIN, D), jnp.float32),
            pltpu.SemaphoreType.DMA,
            pltpu.SemaphoreType.DMA,
            pltpu.SemaphoreType.DMA,
            pltpu.SemaphoreType.DMA,
        ],
    )(_conv_body)
    return k(uf3, uf_tail)


def _dot_body(uid_hbm, aid_hbm, uf_hbm, gf_hbm, ub_hbm, gb_hbm, out_hbm,
              uid_v, aid_v, urow_v, grow_v, ubias_v, gbias_v, out_v,
              sem_u, sem_g, sem_ub, sem_gb):
    wid = lax.axis_index("s") * _NC + lax.axis_index("c")
    base = wid * B_PER_W

    pltpu.sync_copy(uid_hbm.at[pl.ds(base, B_PER_W)], uid_v)
    pltpu.sync_copy(aid_hbm.at[pl.ds(base, B_PER_W)], aid_v)

    copies = [
        pltpu.async_copy(uf_hbm.at[uid_v], urow_v, sem_u),
        pltpu.async_copy(gf_hbm.at[aid_v], grow_v, sem_g),
        pltpu.async_copy(ub_hbm.at[uid_v], ubias_v, sem_ub),
        pltpu.async_copy(gb_hbm.at[aid_v], gbias_v, sem_gb),
    ]
    for cp in copies:
        cp.wait()

    lanes = lax.iota(jnp.int32, L)

    def group(grp, carry):
        row0 = grp * L
        row_idx = lanes + row0
        acc = ubias_v[pl.ds(row0, L)] + gbias_v[pl.ds(row0, L)]
        for j in range(D):
            col_idx = jnp.full((L,), j, jnp.int32)
            u = plsc.load_gather(urow_v, [row_idx, col_idx])
            g = plsc.load_gather(grow_v, [row_idx, col_idx])
            acc = acc + u * g
        out_v[pl.ds(row0, L)] = jnp.maximum(acc, 0.0)
        return carry

    lax.fori_loop(0, N_GROUPS, group, 0)
    pltpu.sync_copy(out_v, out_hbm.at[pl.ds(base, B_PER_W)])


@jax.jit
def _run(user_ids, app_ids, urows, game_factors, user_bias, game_bias):
    mesh = plsc.VectorSubcoreMesh(core_axis_name="c", subcore_axis_name="s")
    k = functools.partial(
        pl.kernel,
        mesh=mesh,
        out_type=jax.ShapeDtypeStruct((B,), jnp.float32),
        compiler_params=pltpu.CompilerParams(
            use_tc_tiling_on_sc=False,
            needs_layout_passes=False,
        ),
        scratch_types=[
            pltpu.VMEM((B_PER_W,), jnp.int32),
            pltpu.VMEM((B_PER_W,), jnp.int32),
            pltpu.VMEM((B_PER_W, D), jnp.float32),
            pltpu.VMEM((B_PER_W, D), jnp.float32),
            pltpu.VMEM((B_PER_W,), jnp.float32),
            pltpu.VMEM((B_PER_W,), jnp.float32),
            pltpu.VMEM((B_PER_W,), jnp.float32),
            pltpu.SemaphoreType.DMA,
            pltpu.SemaphoreType.DMA,
            pltpu.SemaphoreType.DMA,
            pltpu.SemaphoreType.DMA,
        ],
    )(_dot_body)
    return k(user_ids, app_ids, urows, game_factors, user_bias, game_bias)


def kernel(user_ids, app_ids, user_factors, game_factors, user_bias, game_bias):
    uf3 = user_factors.T.reshape(4, 8, N_USERS)
    uf_tail = jnp.concatenate(
        [user_factors[TAIL0:], jnp.zeros((128 - N_TAIL, D), jnp.float32)])
    urows = _convert(uf3, uf_tail)
    return _run(user_ids, app_ids, urows, game_factors, user_bias, game_bias)


# speedup vs baseline: 1.5659x; 1.4731x over previous
import functools

import jax
import jax.numpy as jnp
from jax import lax
from jax.experimental import pallas as pl
from jax.experimental.pallas import tpu as pltpu
from jax.experimental.pallas import tpu_sc as plsc

B = 16384
D = 32
L = 16
N_USERS = 1000000
N_TAIL = N_USERS % 128
TAIL0 = N_USERS - N_TAIL
U_PAD = TAIL0 + 128

_info = plsc.get_sparse_core_info()
_NC, _NS = _info.num_cores, _info.num_subcores
NW = _NC * _NS
B_PER_W = B // NW
N_GROUPS = B_PER_W // L

WIN = 256
LANES_PER_W = 31232
N_WIN = 124


def _conv_body(uf3_hbm, uf_tail_hbm, urows_hbm,
               sbuf0, sbuf1, sbuf2, wbuf0, wbuf1,
               sin0, sin1, sin2, sout0, sout1):
    wid = lax.axis_index("s") * _NC + lax.axis_index("c")
    lane_base = wid * LANES_PER_W

    sbufs = (sbuf0, sbuf1, sbuf2)
    wbufs = (wbuf0, wbuf1)
    sins = (sin0, sin1, sin2)
    souts = (sout0, sout1)

    lanes = lax.iota(jnp.int32, L)

    def start_in(win, slot):
        lane0 = pl.multiple_of(lane_base + win * WIN, 128)
        return pltpu.async_copy(
            uf3_hbm.at[slice(None), slice(None), pl.ds(lane0, WIN)],
            sbufs[slot], sins[slot])

    def transpose_window(islot, oslot):
        sbuf = sbufs[islot]
        wbuf = wbufs[oslot]

        def group(g, carry):
            rows = g * L + lanes
            flat0 = rows * D
            for c in range(D):
                v = plsc.load_gather(
                    sbuf,
                    [jnp.full((L,), c // 8, jnp.int32),
                     jnp.full((L,), c % 8, jnp.int32),
                     rows])
                plsc.store_scatter(wbuf, [flat0 + c], v)
            return carry

        lax.fori_loop(0, WIN // L, group, 0)

    def start_out(win, slot):
        lane0 = pl.multiple_of(lane_base + win * WIN, 128)
        return pltpu.async_copy(
            wbufs[slot], urows_hbm.at[pl.ds(lane0 * D, WIN * D)],
            souts[slot])

    def drain_in(slot):
        pltpu.make_async_copy(
            uf3_hbm.at[slice(None), slice(None), pl.ds(0, WIN)],
            sbufs[slot], sins[slot]).wait()

    def drain_out(slot):
        pltpu.make_async_copy(
            wbufs[slot], urows_hbm.at[pl.ds(0, WIN * D)], souts[slot]).wait()

    start_in(0, 0)
    start_in(1, 1)
    start_in(2, 2)

    def six(p, carry):
        for b in range(6):
            win = 6 * p + b
            islot = b % 3
            oslot = b % 2
            drain_in(islot)

            @pl.when(win >= 2)
            def _():
                drain_out(oslot)

            transpose_window(islot, oslot)
            start_out(win, oslot)

            @pl.when(win + 3 < N_WIN)
            def _():
                start_in(win + 3, islot)
        return carry

    lax.fori_loop(0, N_WIN // 6, six, 0)

    for win in range(6 * (N_WIN // 6), N_WIN):
        islot = win % 3
        oslot = win % 2
        drain_in(islot)
        drain_out(oslot)
        transpose_window(islot, oslot)
        start_out(win, oslot)
        if win + 3 < N_WIN:
            start_in(win + 3, islot)

    drain_out(0)
    drain_out(1)

    @pl.when(wid == 0)
    def _():
        pltpu.sync_copy(uf_tail_hbm, wbuf0.at[pl.ds(0, 128 * D)])
        pltpu.sync_copy(wbuf0.at[pl.ds(0, 128 * D)],
                        urows_hbm.at[pl.ds(TAIL0 * D, 128 * D)])


@functools.partial(jax.jit, donate_argnums=())
def _convert(uf3, uf_tail):
    mesh = plsc.VectorSubcoreMesh(core_axis_name="c", subcore_axis_name="s")
    k = functools.partial(
        pl.kernel,
        mesh=mesh,
        out_type=jax.ShapeDtypeStruct((U_PAD * D,), jnp.float32),
        compiler_params=pltpu.CompilerParams(
            needs_layout_passes=False,
        ),
        scratch_types=[
            pltpu.VMEM((4, 8, WIN), jnp.float32),
            pltpu.VMEM((4, 8, WIN), jnp.float32),
            pltpu.VMEM((4, 8, WIN), jnp.float32),
            pltpu.VMEM((WIN * D,), jnp.float32),
            pltpu.VMEM((WIN * D,), jnp.float32),
            pltpu.SemaphoreType.DMA,
            pltpu.SemaphoreType.DMA,
            pltpu.SemaphoreType.DMA,
            pltpu.SemaphoreType.DMA,
            pltpu.SemaphoreType.DMA,
        ],
    )(_conv_body)
    return k(uf3, uf_tail)


def _dot_body(uid_hbm, aid_hbm, uf_hbm, gf_hbm, ub_hbm, gb_hbm, out_hbm,
              uid_v, aid_v, urow_v, grow_v, ubias_v, gbias_v, out_v,
              sem_u, sem_g, sem_ub, sem_gb):
    wid = lax.axis_index("s") * _NC + lax.axis_index("c")
    base = wid * B_PER_W

    pltpu.sync_copy(uid_hbm.at[pl.ds(base, B_PER_W)], uid_v)
    pltpu.sync_copy(aid_hbm.at[pl.ds(base, B_PER_W)], aid_v)

    copies = [
        pltpu.async_copy(uf_hbm.at[uid_v], urow_v, sem_u),
        pltpu.async_copy(gf_hbm.at[aid_v], grow_v, sem_g),
        pltpu.async_copy(ub_hbm.at[uid_v], ubias_v, sem_ub),
        pltpu.async_copy(gb_hbm.at[aid_v], gbias_v, sem_gb),
    ]
    for cp in copies:
        cp.wait()

    lanes = lax.iota(jnp.int32, L)

    def group(grp, carry):
        row0 = grp * L
        row_idx = lanes + row0
        acc = ubias_v[pl.ds(row0, L)] + gbias_v[pl.ds(row0, L)]
        for j in range(D):
            col_idx = jnp.full((L,), j, jnp.int32)
            u = plsc.load_gather(urow_v, [row_idx, col_idx])
            g = plsc.load_gather(grow_v, [row_idx, col_idx])
            acc = acc + u * g
        out_v[pl.ds(row0, L)] = jnp.maximum(acc, 0.0)
        return carry

    lax.fori_loop(0, N_GROUPS, group, 0)
    pltpu.sync_copy(out_v, out_hbm.at[pl.ds(base, B_PER_W)])


@jax.jit
def _run(user_ids, app_ids, urows, game_factors, user_bias, game_bias):
    mesh = plsc.VectorSubcoreMesh(core_axis_name="c", subcore_axis_name="s")
    k = functools.partial(
        pl.kernel,
        mesh=mesh,
        out_type=jax.ShapeDtypeStruct((B,), jnp.float32),
        compiler_params=pltpu.CompilerParams(
            use_tc_tiling_on_sc=False,
            needs_layout_passes=False,
        ),
        scratch_types=[
            pltpu.VMEM((B_PER_W,), jnp.int32),
            pltpu.VMEM((B_PER_W,), jnp.int32),
            pltpu.VMEM((B_PER_W, D), jnp.float32),
            pltpu.VMEM((B_PER_W, D), jnp.float32),
            pltpu.VMEM((B_PER_W,), jnp.float32),
            pltpu.VMEM((B_PER_W,), jnp.float32),
            pltpu.VMEM((B_PER_W,), jnp.float32),
            pltpu.SemaphoreType.DMA,
            pltpu.SemaphoreType.DMA,
            pltpu.SemaphoreType.DMA,
            pltpu.SemaphoreType.DMA,
        ],
    )(_dot_body)
    return k(user_ids, app_ids, urows, game_factors, user_bias, game_bias)


def kernel(user_ids, app_ids, user_factors, game_factors, user_bias, game_bias):
    uf3 = user_factors.T.reshape(4, 8, N_USERS)
    uf_tail = jnp.concatenate(
        [user_factors[TAIL0:], jnp.zeros((128 - N_TAIL, D), jnp.float32)])
    urows = _convert(uf3, uf_tail.reshape(-1)).reshape(U_PAD, D)
    return _run(user_ids, app_ids, urows, game_factors, user_bias, game_bias)


# speedup vs baseline: 1.5738x; 1.0051x over previous
import functools

import jax
import jax.numpy as jnp
from jax import lax
from jax.experimental import pallas as pl
from jax.experimental.pallas import tpu as pltpu
from jax.experimental.pallas import tpu_sc as plsc

B = 16384
D = 32
L = 16
N_USERS = 1000000
N_TAIL = N_USERS % 128
TAIL0 = N_USERS - N_TAIL
U_PAD = TAIL0 + 128

_info = plsc.get_sparse_core_info()
_NC, _NS = _info.num_cores, _info.num_subcores
NW = _NC * _NS
B_PER_W = B // NW
N_GROUPS = B_PER_W // L

WIN = 512
LANES_PER_W = 31232
N_WIN = 62


def _conv_body(uf3_hbm, uf_tail_hbm, urows_hbm,
               sbuf0, sbuf1, wbuf0, wbuf1,
               sin0, sin1, sout0, sout1):
    wid = lax.axis_index("s") * _NC + lax.axis_index("c")
    lane_base = wid * LANES_PER_W

    sbufs = (sbuf0, sbuf1)
    wbufs = (wbuf0, wbuf1)
    sins = (sin0, sin1)
    souts = (sout0, sout1)

    lanes = lax.iota(jnp.int32, L)

    def start_in(win, slot):
        lane0 = pl.multiple_of(lane_base + win * WIN, 128)
        return pltpu.async_copy(
            uf3_hbm.at[slice(None), slice(None), pl.ds(lane0, WIN)],
            sbufs[slot], sins[slot])

    def transpose_window(islot, oslot):
        sbuf = sbufs[islot]
        wbuf = wbufs[oslot]

        def group(g, carry):
            rows = g * L + lanes
            flat0 = rows * D
            for c in range(D):
                v = plsc.load_gather(
                    sbuf,
                    [jnp.full((L,), c // 8, jnp.int32),
                     jnp.full((L,), c % 8, jnp.int32),
                     rows])
                plsc.store_scatter(wbuf, [flat0 + c], v)
            return carry

        lax.fori_loop(0, WIN // L, group, 0)

    def start_out(win, slot):
        lane0 = pl.multiple_of(lane_base + win * WIN, 128)
        return pltpu.async_copy(
            wbufs[slot], urows_hbm.at[pl.ds(lane0 * D, WIN * D)],
            souts[slot])

    def drain_in(slot):
        pltpu.make_async_copy(
            uf3_hbm.at[slice(None), slice(None), pl.ds(0, WIN)],
            sbufs[slot], sins[slot]).wait()

    def drain_out(slot):
        pltpu.make_async_copy(
            wbufs[slot], urows_hbm.at[pl.ds(0, WIN * D)], souts[slot]).wait()

    start_in(0, 0)
    start_in(1, 1)

    def six(p, carry):
        for b in range(2):
            win = 2 * p + b
            islot = b
            oslot = b
            drain_in(islot)

            @pl.when(win >= 2)
            def _():
                drain_out(oslot)

            transpose_window(islot, oslot)
            start_out(win, oslot)

            @pl.when(win + 2 < N_WIN)
            def _():
                start_in(win + 2, islot)
        return carry

    lax.fori_loop(0, N_WIN // 2, six, 0)
    drain_out(0)
    drain_out(1)

    @pl.when(wid == 0)
    def _():
        pltpu.sync_copy(uf_tail_hbm, wbuf0.at[pl.ds(0, 128 * D)])
        pltpu.sync_copy(wbuf0.at[pl.ds(0, 128 * D)],
                        urows_hbm.at[pl.ds(TAIL0 * D, 128 * D)])


@functools.partial(jax.jit, donate_argnums=())
def _convert(uf3, uf_tail):
    mesh = plsc.VectorSubcoreMesh(core_axis_name="c", subcore_axis_name="s")
    k = functools.partial(
        pl.kernel,
        mesh=mesh,
        out_type=jax.ShapeDtypeStruct((U_PAD * D,), jnp.float32),
        compiler_params=pltpu.CompilerParams(
            needs_layout_passes=False,
        ),
        scratch_types=[
            pltpu.VMEM((4, 8, WIN), jnp.float32),
            pltpu.VMEM((4, 8, WIN), jnp.float32),
            pltpu.VMEM((WIN * D,), jnp.float32),
            pltpu.VMEM((WIN * D,), jnp.float32),
            pltpu.SemaphoreType.DMA,
            pltpu.SemaphoreType.DMA,
            pltpu.SemaphoreType.DMA,
            pltpu.SemaphoreType.DMA,
        ],
    )(_conv_body)
    return k(uf3, uf_tail)


def _dot_body(uid_hbm, aid_hbm, uf_hbm, gf_hbm, ub_hbm, gb_hbm, out_hbm,
              uid_v, aid_v, urow_v, grow_v, ubias_v, gbias_v, out_v,
              sem_u, sem_g, sem_ub, sem_gb):
    wid = lax.axis_index("s") * _NC + lax.axis_index("c")
    base = wid * B_PER_W

    pltpu.sync_copy(uid_hbm.at[pl.ds(base, B_PER_W)], uid_v)
    pltpu.sync_copy(aid_hbm.at[pl.ds(base, B_PER_W)], aid_v)

    cp_ub = pltpu.async_copy(ub_hbm.at[uid_v], ubias_v, sem_ub)
    cp_gb = pltpu.async_copy(gb_hbm.at[aid_v], gbias_v, sem_gb)

    lanes = lax.iota(jnp.int32, L)
    HALF = B_PER_W // 2

    for h in range(2):
        cp_u = pltpu.async_copy(
            uf_hbm.at[uid_v.at[pl.ds(h * HALF, HALF)]], urow_v, sem_u)
        cp_g = pltpu.async_copy(
            gf_hbm.at[aid_v.at[pl.ds(h * HALF, HALF)]], grow_v, sem_g)
        cp_u.wait()
        cp_g.wait()
        if h == 0:
            cp_ub.wait()
            cp_gb.wait()

        def group(grp, carry):
            row0 = grp * L
            row_idx = lanes + row0
            b0 = h * HALF + row0
            acc = ubias_v[pl.ds(b0, L)] + gbias_v[pl.ds(b0, L)]
            for j in range(D):
                col_idx = jnp.full((L,), j, jnp.int32)
                u = plsc.load_gather(urow_v, [row_idx, col_idx])
                g = plsc.load_gather(grow_v, [row_idx, col_idx])
                acc = acc + u * g
            out_v[pl.ds(b0, L)] = jnp.maximum(acc, 0.0)
            return carry

        lax.fori_loop(0, HALF // L, group, 0)

    pltpu.sync_copy(out_v, out_hbm.at[pl.ds(base, B_PER_W)])


@jax.jit
def _run(user_ids, app_ids, urows, game_factors, user_bias, game_bias):
    mesh = plsc.VectorSubcoreMesh(core_axis_name="c", subcore_axis_name="s")
    k = functools.partial(
        pl.kernel,
        mesh=mesh,
        out_type=jax.ShapeDtypeStruct((B,), jnp.float32),
        compiler_params=pltpu.CompilerParams(
            use_tc_tiling_on_sc=False,
            needs_layout_passes=False,
        ),
        scratch_types=[
            pltpu.VMEM((B_PER_W,), jnp.int32),
            pltpu.VMEM((B_PER_W,), jnp.int32),
            pltpu.VMEM((B_PER_W // 2, D), jnp.float32),
            pltpu.VMEM((B_PER_W // 2, D), jnp.float32),
            pltpu.VMEM((B_PER_W,), jnp.float32),
            pltpu.VMEM((B_PER_W,), jnp.float32),
            pltpu.VMEM((B_PER_W,), jnp.float32),
            pltpu.SemaphoreType.DMA,
            pltpu.SemaphoreType.DMA,
            pltpu.SemaphoreType.DMA,
            pltpu.SemaphoreType.DMA,
        ],
    )(_dot_body)
    return k(user_ids, app_ids, urows, game_factors, user_bias, game_bias)


def kernel(user_ids, app_ids, user_factors, game_factors, user_bias, game_bias):
    uf3 = user_factors.T.reshape(4, 8, N_USERS)
    uf_tail = jnp.concatenate(
        [user_factors[TAIL0:], jnp.zeros((128 - N_TAIL, D), jnp.float32)])
    urows = _convert(uf3, uf_tail.reshape(-1)).reshape(U_PAD, D)
    return _run(user_ids, app_ids, urows, game_factors, user_bias, game_bias)


# speedup vs baseline: 2.0116x; 1.2782x over previous
import functools

import jax
import jax.numpy as jnp
from jax import lax
from jax.experimental import pallas as pl
from jax.experimental.pallas import tpu as pltpu
from jax.experimental.pallas import tpu_sc as plsc

B = 16384
D = 32
L = 16
N_USERS = 1000000
N_TAIL = N_USERS % 128
TAIL0 = N_USERS - N_TAIL
U_PAD = TAIL0 + 128

_info = plsc.get_sparse_core_info()
_NC, _NS = _info.num_cores, _info.num_subcores
NW = _NC * _NS
B_PER_W = B // NW
N_GROUPS = B_PER_W // L

WIN = 512
LANES_PER_W = 31232
N_WIN = 62


def _conv_body(uf3_hbm, uf_tail_hbm, urows_hbm,
               sbuf0, sbuf1, wbuf0, wbuf1,
               sin0, sin1, sout0, sout1):
    wid = lax.axis_index("s") * _NC + lax.axis_index("c")
    lane_base = wid * LANES_PER_W

    sbufs = (sbuf0, sbuf1)
    wbufs = (wbuf0, wbuf1)
    sins = (sin0, sin1)
    souts = (sout0, sout1)

    lanes = lax.iota(jnp.int32, L)

    def start_in(win, slot):
        lane0 = pl.multiple_of(lane_base + win * WIN, 128)
        return pltpu.async_copy(
            uf3_hbm.at[slice(None), slice(None), pl.ds(lane0, WIN)],
            sbufs[slot], sins[slot])

    def transpose_window(islot, oslot):
        sbuf = sbufs[islot]
        wbuf = wbufs[oslot]

        def group(g, carry):
            rows = g * L + lanes
            flat0 = rows * D
            vs = [
                plsc.load_gather(
                    sbuf,
                    [jnp.full((L,), c // 8, jnp.int32),
                     jnp.full((L,), c % 8, jnp.int32),
                     rows])
                for c in range(D)
            ]
            for c in range(D):
                plsc.store_scatter(wbuf, [flat0 + c], vs[c])
            return carry

        lax.fori_loop(0, WIN // L, group, 0)

    def start_out(win, slot):
        lane0 = pl.multiple_of(lane_base + win * WIN, 128)
        return pltpu.async_copy(
            wbufs[slot], urows_hbm.at[pl.ds(lane0 * D, WIN * D)],
            souts[slot])

    def drain_in(slot):
        pltpu.make_async_copy(
            uf3_hbm.at[slice(None), slice(None), pl.ds(0, WIN)],
            sbufs[slot], sins[slot]).wait()

    def drain_out(slot):
        pltpu.make_async_copy(
            wbufs[slot], urows_hbm.at[pl.ds(0, WIN * D)], souts[slot]).wait()

    start_in(0, 0)
    start_in(1, 1)

    def six(p, carry):
        for b in range(2):
            win = 2 * p + b
            islot = b
            oslot = b
            drain_in(islot)

            @pl.when(win >= 2)
            def _():
                drain_out(oslot)

            transpose_window(islot, oslot)
            start_out(win, oslot)

            @pl.when(win + 2 < N_WIN)
            def _():
                start_in(win + 2, islot)
        return carry

    lax.fori_loop(0, N_WIN // 2, six, 0)
    drain_out(0)
    drain_out(1)

    @pl.when(wid == 0)
    def _():
        pltpu.sync_copy(uf_tail_hbm, wbuf0.at[pl.ds(0, 128 * D)])
        pltpu.sync_copy(wbuf0.at[pl.ds(0, 128 * D)],
                        urows_hbm.at[pl.ds(TAIL0 * D, 128 * D)])


@functools.partial(jax.jit, donate_argnums=())
def _convert(uf3, uf_tail):
    mesh = plsc.VectorSubcoreMesh(core_axis_name="c", subcore_axis_name="s")
    k = functools.partial(
        pl.kernel,
        mesh=mesh,
        out_type=jax.ShapeDtypeStruct((U_PAD * D,), jnp.float32),
        compiler_params=pltpu.CompilerParams(
            needs_layout_passes=False,
        ),
        scratch_types=[
            pltpu.VMEM((4, 8, WIN), jnp.float32),
            pltpu.VMEM((4, 8, WIN), jnp.float32),
            pltpu.VMEM((WIN * D,), jnp.float32),
            pltpu.VMEM((WIN * D,), jnp.float32),
            pltpu.SemaphoreType.DMA,
            pltpu.SemaphoreType.DMA,
            pltpu.SemaphoreType.DMA,
            pltpu.SemaphoreType.DMA,
        ],
    )(_conv_body)
    return k(uf3, uf_tail)


def _dot_body(uid_hbm, aid_hbm, uf_hbm, gf_hbm, ub_hbm, gb_hbm, out_hbm,
              uid_v, aid_v, urow_v, grow_v, ubias_v, gbias_v, out_v,
              sem_u, sem_g, sem_ub, sem_gb):
    wid = lax.axis_index("s") * _NC + lax.axis_index("c")
    base = wid * B_PER_W

    pltpu.sync_copy(uid_hbm.at[pl.ds(base, B_PER_W)], uid_v)
    pltpu.sync_copy(aid_hbm.at[pl.ds(base, B_PER_W)], aid_v)

    cp_ub = pltpu.async_copy(ub_hbm.at[uid_v], ubias_v, sem_ub)
    cp_gb = pltpu.async_copy(gb_hbm.at[aid_v], gbias_v, sem_gb)

    lanes = lax.iota(jnp.int32, L)
    HALF = B_PER_W // 2

    for h in range(2):
        cp_u = pltpu.async_copy(
            uf_hbm.at[uid_v.at[pl.ds(h * HALF, HALF)]], urow_v, sem_u)
        cp_g = pltpu.async_copy(
            gf_hbm.at[aid_v.at[pl.ds(h * HALF, HALF)]], grow_v, sem_g)
        cp_u.wait()
        cp_g.wait()
        if h == 0:
            cp_ub.wait()
            cp_gb.wait()

        def group(grp, carry):
            row0 = grp * L
            row_idx = lanes + row0
            b0 = h * HALF + row0
            acc = ubias_v[pl.ds(b0, L)] + gbias_v[pl.ds(b0, L)]
            for j in range(D):
                col_idx = jnp.full((L,), j, jnp.int32)
                u = plsc.load_gather(urow_v, [row_idx, col_idx])
                g = plsc.load_gather(grow_v, [row_idx, col_idx])
                acc = acc + u * g
            out_v[pl.ds(b0, L)] = jnp.maximum(acc, 0.0)
            return carry

        lax.fori_loop(0, HALF // L, group, 0)

    pltpu.sync_copy(out_v, out_hbm.at[pl.ds(base, B_PER_W)])


@jax.jit
def _run(user_ids, app_ids, urows, game_factors, user_bias, game_bias):
    mesh = plsc.VectorSubcoreMesh(core_axis_name="c", subcore_axis_name="s")
    k = functools.partial(
        pl.kernel,
        mesh=mesh,
        out_type=jax.ShapeDtypeStruct((B,), jnp.float32),
        compiler_params=pltpu.CompilerParams(
            use_tc_tiling_on_sc=False,
            needs_layout_passes=False,
        ),
        scratch_types=[
            pltpu.VMEM((B_PER_W,), jnp.int32),
            pltpu.VMEM((B_PER_W,), jnp.int32),
            pltpu.VMEM((B_PER_W // 2, D), jnp.float32),
            pltpu.VMEM((B_PER_W // 2, D), jnp.float32),
            pltpu.VMEM((B_PER_W,), jnp.float32),
            pltpu.VMEM((B_PER_W,), jnp.float32),
            pltpu.VMEM((B_PER_W,), jnp.float32),
            pltpu.SemaphoreType.DMA,
            pltpu.SemaphoreType.DMA,
            pltpu.SemaphoreType.DMA,
            pltpu.SemaphoreType.DMA,
        ],
    )(_dot_body)
    return k(user_ids, app_ids, urows, game_factors, user_bias, game_bias)


def kernel(user_ids, app_ids, user_factors, game_factors, user_bias, game_bias):
    uf3 = user_factors.T.reshape(4, 8, N_USERS)
    uf_tail = jnp.concatenate(
        [user_factors[TAIL0:], jnp.zeros((128 - N_TAIL, D), jnp.float32)])
    urows = _convert(uf3, uf_tail.reshape(-1)).reshape(U_PAD, D)
    return _run(user_ids, app_ids, urows, game_factors, user_bias, game_bias)
